# Initial kernel scaffold; baseline (speedup 1.0000x reference)
import functools

import numpy as np
import jax
import jax.numpy as jnp
from jax import lax
from jax.experimental import pallas as pl
from jax.experimental.pallas import tpu as pltpu

B, C, N, K = 2, 256, 2048, 16
G = 4
LC, NLC = 192, 64
GC = LC // G
NCH = NLC // G
NK = N * K
NB = 8
nb = N // NB
NKb = nb * K


def _stage1_body(x_ref, idxt_ref, absx_ref, wq_ref, wk_ref, wv_ref,
                 wqn_ref, wkn_ref, wvn_ref, sel_ref,
                 outl_ref, attn_ref, idxs_ref, qn_ref, kn_ref, vn_ref):
    x2 = x_ref[0]
    sel = sel_ref[...]
    q = jnp.dot(wq_ref[...], x2, preferred_element_type=jnp.float32)
    k = jnp.dot(wk_ref[...], x2, preferred_element_type=jnp.float32)
    v = jnp.dot(wv_ref[...], x2, preferred_element_type=jnp.float32)
    prod = q * k
    out = jnp.concatenate(
        [jnp.sum(prod[g * GC:(g + 1) * GC], axis=0, keepdims=True)
         for g in range(G)], axis=0)
    e = jnp.exp(out)
    den = jnp.dot(e, sel, preferred_element_type=jnp.float32)
    den_rep = lax.dot_general(den, sel, (((1,), (1,)), ((), ())),
                              preferred_element_type=jnp.float32)
    sm = e / den_rep
    attn_ref[0] = sm
    w = (v.reshape(G, GC, NKb) * sm[:, None, :]).reshape(LC, NKb)
    outl_ref[0] = jnp.dot(w, sel, preferred_element_type=jnp.float32)
    idxt = idxt_ref[0]
    rows = []
    for i in range(K):
        dup = None
        for j in range(i + 1, K):
            eq = idxt[i:i + 1] == idxt[j:j + 1]
            dup = eq if dup is None else (dup | eq)
        row = idxt[i:i + 1]
        if dup is not None:
            row = jnp.where(dup, N + i, row)
        rows.append(row)
    idxs_ref[0] = jnp.concatenate(rows, axis=0)
    a = absx_ref[0]
    qn_ref[0] = jnp.dot(wqn_ref[...], a, preferred_element_type=jnp.float32)
    kn_ref[0] = jnp.dot(wkn_ref[...], a, preferred_element_type=jnp.float32)
    vn_ref[0] = jnp.dot(wvn_ref[...], a, preferred_element_type=jnp.float32)


def _stage1(xr, idx_t, absr, Wq, Wk, Wv, Wq_nl, Wk_nl, Wv_nl, sel):
    f32 = jnp.float32
    return pl.pallas_call(
        _stage1_body,
        grid=(B, NB),
        in_specs=[
            pl.BlockSpec((1, C, NKb), lambda b, n: (b, 0, n)),
            pl.BlockSpec((1, K, nb), lambda b, n: (b, 0, n)),
            pl.BlockSpec((1, C // 2, nb), lambda b, n: (b, 0, n)),
            pl.BlockSpec((LC, C), lambda b, n: (0, 0)),
            pl.BlockSpec((LC, C), lambda b, n: (0, 0)),
            pl.BlockSpec((LC, C), lambda b, n: (0, 0)),
            pl.BlockSpec((NLC, C // 2), lambda b, n: (0, 0)),
            pl.BlockSpec((NLC, C // 2), lambda b, n: (0, 0)),
            pl.BlockSpec((NLC, C // 2), lambda b, n: (0, 0)),
            pl.BlockSpec((NKb, nb), lambda b, n: (0, 0)),
        ],
        out_specs=[
            pl.BlockSpec((1, LC, nb), lambda b, n: (b, 0, n)),
            pl.BlockSpec((1, G, NKb), lambda b, n: (b, 0, n)),
            pl.BlockSpec((1, K, nb), lambda b, n: (b, 0, n)),
            pl.BlockSpec((1, NLC, nb), lambda b, n: (b, 0, n)),
            pl.BlockSpec((1, NLC, nb), lambda b, n: (b, 0, n)),
            pl.BlockSpec((1, NLC, nb), lambda b, n: (b, 0, n)),
        ],
        out_shape=[
            jax.ShapeDtypeStruct((B, LC, N), f32),
            jax.ShapeDtypeStruct((B, G, NK), f32),
            jax.ShapeDtypeStruct((B, K, N), jnp.int32),
            jax.ShapeDtypeStruct((B, NLC, N), f32),
            jax.ShapeDtypeStruct((B, NLC, N), f32),
            jax.ShapeDtypeStruct((B, NLC, N), f32),
        ],
    )(xr, idx_t, absr, Wq, Wk, Wv, Wq_nl, Wk_nl, Wv_nl, sel)


CNK = 512
NCHUNK = NK // CNK


def _stage2_body(attn_ref, idxf_ref, score_ref):
    ch = pl.program_id(1)
    idx_row = idxf_ref[0]
    idx_col = jnp.transpose(idx_row.reshape(1, CNK), (1, 0))
    iota_m = lax.broadcasted_iota(jnp.int32, (CNK, N), 1)
    oh = (iota_m == idx_col).astype(jnp.float32)
    part = jnp.dot(attn_ref[0], oh, preferred_element_type=jnp.float32)

    @pl.when(ch == 0)
    def _():
        score_ref[0] = part

    @pl.when(ch != 0)
    def _():
        score_ref[0] = score_ref[0] + part


def _stage2(attn_flat, idx_flat):
    return pl.pallas_call(
        _stage2_body,
        grid=(B, NCHUNK),
        in_specs=[
            pl.BlockSpec((1, G, CNK), lambda b, ch: (b, 0, ch)),
            pl.BlockSpec((1, 1, CNK), lambda b, ch: (b, 0, ch)),
        ],
        out_specs=pl.BlockSpec((1, G, N), lambda b, ch: (b, 0, 0)),
        out_shape=jax.ShapeDtypeStruct((B, G, N), jnp.float32),
    )(attn_flat, idx_flat)


def _stage3_body(score_ref, qn_ref, kn_ref, vn_ref, out_ref):
    s = score_ref[0, 0].reshape(1, N)
    iota = lax.broadcasted_iota(jnp.int32, (1, N), 1)
    oh_rows, vals = [], []
    for _ in range(K):
        m = jnp.max(s)
        j = jnp.min(jnp.where(s == m, iota, N))
        hit = iota == j
        oh_rows.append(hit.astype(jnp.float32))
        vals.append(m.reshape(1, 1))
        s = jnp.where(hit, -3e38, s)
    oh = jnp.concatenate(oh_rows, axis=0)
    val = jnp.concatenate(vals, axis=0)
    qn = qn_ref[0, 0]
    kn = kn_ref[0, 0]
    vn = vn_ref[0, 0]
    k_gT = lax.dot_general(oh, kn, (((1,), (1,)), ((), ())),
                           preferred_element_type=jnp.float32)
    v_gT = lax.dot_general(oh, vn, (((1,), (1,)), ((), ())),
                           preferred_element_type=jnp.float32)
    v_gT = v_gT * jnp.tanh(val)
    at = jnp.dot(k_gT, qn, preferred_element_type=jnp.float32)
    at = at - jnp.max(at, axis=0, keepdims=True)
    e = jnp.exp(at)
    at_sm = e / jnp.sum(e, axis=0, keepdims=True)
    out_ref[0, 0] = lax.dot_general(
        v_gT, at_sm, (((0,), (0,)), ((), ())),
        preferred_element_type=jnp.float32)


def _stage3(score, qn, kn, vn):
    return pl.pallas_call(
        _stage3_body,
        grid=(B, G),
        in_specs=[
            pl.BlockSpec((1, 1, N), lambda b, g: (b, g, 0)),
            pl.BlockSpec((1, 1, NCH, N), lambda b, g: (b, g, 0, 0)),
            pl.BlockSpec((1, 1, NCH, N), lambda b, g: (b, g, 0, 0)),
            pl.BlockSpec((1, 1, NCH, N), lambda b, g: (b, g, 0, 0)),
        ],
        out_specs=pl.BlockSpec((1, 1, NCH, N), lambda b, g: (b, g, 0, 0)),
        out_shape=jax.ShapeDtypeStruct((B, G, NCH, N), jnp.float32),
    )(score, qn, kn, vn)


def kernel(x, abs_x, idx, Wq, Wk, Wv, Wq_nl, Wk_nl, Wv_nl):
    xr = x.reshape(B, C, NK)
    idx_t = jnp.swapaxes(idx.reshape(B, N, K), 1, 2)
    absr = abs_x.reshape(B, C // 2, N)
    sel = jnp.asarray(np.repeat(np.eye(nb, dtype=np.float32), K, axis=0))
    out_l, attn, idx_safe_t, qn, kn, vn = _stage1(
        xr, idx_t, absr, Wq, Wk, Wv, Wq_nl, Wk_nl, Wv_nl, sel)
    idx_flat = jnp.swapaxes(idx_safe_t, 1, 2).reshape(B, 1, NK)
    score = _stage2(attn, idx_flat)
    out_all = _stage3(score,
                      qn.reshape(B, G, NCH, N),
                      kn.reshape(B, G, NCH, N),
                      vn.reshape(B, G, NCH, N))
    return jnp.concatenate([out_l.reshape(B, LC, N, 1),
                            out_all.reshape(B, NLC, N, 1)], axis=1)


# speedup vs baseline: 2.3670x; 2.3670x over previous
import functools

import numpy as np
import jax
import jax.numpy as jnp
from jax import lax
from jax.experimental import pallas as pl
from jax.experimental.pallas import tpu as pltpu

def _sort_network(n=16):
    comps = []
    length = 1
    while length < n:
        for lo in range(0, n, 2 * length):
            step = length
            while step >= 1:
                if step == length:
                    rng = [(lo + i, lo + 2 * length - 1 - i)
                           for i in range(length)]
                else:
                    rng = [(i, i + step)
                           for i in range(lo, lo + 2 * length - step)
                           if (i - lo) % (2 * step) < step]
                comps.extend(rng)
                step //= 2
        length *= 2
    return comps


_COMPS = _sort_network(16)

B, C, N, K = 2, 256, 2048, 16
G = 4
LC, NLC = 192, 64
GC = LC // G
NCH = NLC // G
NK = N * K
NB = 8
nb = N // NB
NKb = nb * K


def _stage1_body(x_ref, idxt_ref, absx_ref, wq_ref, wk_ref, wv_ref,
                 wqn_ref, wkn_ref, wvn_ref, sel_ref,
                 outl_ref, attn_ref, idxs_ref, qn_ref, kn_ref, vn_ref):
    x2 = x_ref[0]
    sel = sel_ref[...]
    q = jnp.dot(wq_ref[...], x2, preferred_element_type=jnp.float32)
    k = jnp.dot(wk_ref[...], x2, preferred_element_type=jnp.float32)
    v = jnp.dot(wv_ref[...], x2, preferred_element_type=jnp.float32)
    prod = q * k
    out = jnp.concatenate(
        [jnp.sum(prod[g * GC:(g + 1) * GC], axis=0, keepdims=True)
         for g in range(G)], axis=0)
    e = jnp.exp(out)
    den = jnp.dot(e, sel, preferred_element_type=jnp.float32, precision=lax.Precision.HIGHEST)
    den_rep = lax.dot_general(den, sel, (((1,), (1,)), ((), ())),
                              preferred_element_type=jnp.float32, precision=lax.Precision.HIGHEST)
    sm = e / den_rep
    attn_ref[0] = sm
    w = (v.reshape(G, GC, NKb) * sm[:, None, :]).reshape(LC, NKb)
    outl_ref[0] = jnp.dot(w, sel, preferred_element_type=jnp.float32, precision=lax.Precision.HIGHEST)
    idxt = idxt_ref[0]
    a = [idxt[i:i + 1] for i in range(K)]
    p = [jnp.zeros_like(a[0]) + i for i in range(K)]
    for i, l in _COMPS:
        swap = a[i] > a[l]
        a[i], a[l] = (jnp.where(swap, a[l], a[i]),
                      jnp.where(swap, a[i], a[l]))
        p[i], p[l] = (jnp.where(swap, p[l], p[i]),
                      jnp.where(swap, p[i], p[l]))
    ones = jnp.ones_like(a[0], dtype=jnp.bool_)
    keep_s = [(a[t + 1] != a[t]) if t < K - 1 else ones for t in range(K)]
    rows = []
    for i in range(K):
        keep = None
        for t in range(K):
            hit = (p[t] == i) & keep_s[t]
            keep = hit if keep is None else (keep | hit)
        rows.append(jnp.where(keep, idxt[i:i + 1], N + i))
    idxs_ref[0] = jnp.concatenate(rows, axis=0)
    a = absx_ref[0]
    qn_ref[0] = jnp.dot(wqn_ref[...], a, preferred_element_type=jnp.float32, precision=lax.Precision.HIGHEST)
    kn_ref[0] = jnp.dot(wkn_ref[...], a, preferred_element_type=jnp.float32, precision=lax.Precision.HIGHEST)
    vn_ref[0] = jnp.dot(wvn_ref[...], a, preferred_element_type=jnp.float32, precision=lax.Precision.HIGHEST)


def _stage1(xr, idx_t, absr, Wq, Wk, Wv, Wq_nl, Wk_nl, Wv_nl, sel):
    f32 = jnp.float32
    return pl.pallas_call(
        _stage1_body,
        grid=(B, NB),
        in_specs=[
            pl.BlockSpec((1, C, NKb), lambda b, n: (b, 0, n)),
            pl.BlockSpec((1, K, nb), lambda b, n: (b, 0, n)),
            pl.BlockSpec((1, C // 2, nb), lambda b, n: (b, 0, n)),
            pl.BlockSpec((LC, C), lambda b, n: (0, 0)),
            pl.BlockSpec((LC, C), lambda b, n: (0, 0)),
            pl.BlockSpec((LC, C), lambda b, n: (0, 0)),
            pl.BlockSpec((NLC, C // 2), lambda b, n: (0, 0)),
            pl.BlockSpec((NLC, C // 2), lambda b, n: (0, 0)),
            pl.BlockSpec((NLC, C // 2), lambda b, n: (0, 0)),
            pl.BlockSpec((NKb, nb), lambda b, n: (0, 0)),
        ],
        out_specs=[
            pl.BlockSpec((1, LC, nb), lambda b, n: (b, 0, n)),
            pl.BlockSpec((1, G, NKb), lambda b, n: (b, 0, n)),
            pl.BlockSpec((1, K, nb), lambda b, n: (b, 0, n)),
            pl.BlockSpec((1, NLC, nb), lambda b, n: (b, 0, n)),
            pl.BlockSpec((1, NLC, nb), lambda b, n: (b, 0, n)),
            pl.BlockSpec((1, NLC, nb), lambda b, n: (b, 0, n)),
        ],
        out_shape=[
            jax.ShapeDtypeStruct((B, LC, N), f32),
            jax.ShapeDtypeStruct((B, G, NK), f32),
            jax.ShapeDtypeStruct((B, K, N), jnp.int32),
            jax.ShapeDtypeStruct((B, NLC, N), f32),
            jax.ShapeDtypeStruct((B, NLC, N), f32),
            jax.ShapeDtypeStruct((B, NLC, N), f32),
        ],
    )(xr, idx_t, absr, Wq, Wk, Wv, Wq_nl, Wk_nl, Wv_nl, sel)


CNK = 512
NCHUNK = NK // CNK


def _stage2_body(attn_ref, idxf_ref, score_ref):
    ch = pl.program_id(1)
    idx_row = idxf_ref[0]
    idx_col = jnp.transpose(idx_row.reshape(1, CNK), (1, 0))
    iota_m = lax.broadcasted_iota(jnp.int32, (CNK, N), 1)
    oh = (iota_m == idx_col).astype(jnp.float32)
    part = jnp.dot(attn_ref[0], oh, preferred_element_type=jnp.float32, precision=lax.Precision.HIGHEST)

    @pl.when(ch == 0)
    def _():
        score_ref[0] = part

    @pl.when(ch != 0)
    def _():
        score_ref[0] = score_ref[0] + part


def _stage2(attn_flat, idx_flat):
    return pl.pallas_call(
        _stage2_body,
        grid=(B, NCHUNK),
        in_specs=[
            pl.BlockSpec((1, G, CNK), lambda b, ch: (b, 0, ch)),
            pl.BlockSpec((1, 1, CNK), lambda b, ch: (b, 0, ch)),
        ],
        out_specs=pl.BlockSpec((1, G, N), lambda b, ch: (b, 0, 0)),
        out_shape=jax.ShapeDtypeStruct((B, G, N), jnp.float32),
    )(attn_flat, idx_flat)


def _stage3_body(score_ref, qn_ref, kn_ref, vn_ref, out_ref):
    s = score_ref[0, 0]
    iota = lax.broadcasted_iota(jnp.int32, (1, N), 1)
    oh_rows, vals = [], []
    for _ in range(K):
        m = jnp.max(s)
        j = jnp.min(jnp.where(s == m, iota, N))
        hit = iota == j
        oh_rows.append(hit.astype(jnp.float32))
        vals.append(m.reshape(1, 1))
        s = jnp.where(hit, -3e38, s)
    oh = jnp.concatenate(oh_rows, axis=0)
    val = jnp.concatenate(vals, axis=0)
    qn = qn_ref[0, 0]
    kn = kn_ref[0, 0]
    vn = vn_ref[0, 0]
    k_gT = lax.dot_general(oh, kn, (((1,), (1,)), ((), ())),
                           preferred_element_type=jnp.float32, precision=lax.Precision.HIGHEST)
    v_gT = lax.dot_general(oh, vn, (((1,), (1,)), ((), ())),
                           preferred_element_type=jnp.float32, precision=lax.Precision.HIGHEST)
    v_gT = v_gT * jnp.tanh(val)
    at = jnp.dot(k_gT, qn, preferred_element_type=jnp.float32, precision=lax.Precision.HIGHEST)
    at = at - jnp.max(at, axis=0, keepdims=True)
    e = jnp.exp(at)
    at_sm = e / jnp.sum(e, axis=0, keepdims=True)
    out_ref[0, 0] = lax.dot_general(
        v_gT, at_sm, (((0,), (0,)), ((), ())),
        preferred_element_type=jnp.float32, precision=lax.Precision.HIGHEST)


def _stage3(score, qn, kn, vn):
    return pl.pallas_call(
        _stage3_body,
        grid=(B, G),
        in_specs=[
            pl.BlockSpec((1, 1, 1, N), lambda b, g: (b, g, 0, 0)),
            pl.BlockSpec((1, 1, NCH, N), lambda b, g: (b, g, 0, 0)),
            pl.BlockSpec((1, 1, NCH, N), lambda b, g: (b, g, 0, 0)),
            pl.BlockSpec((1, 1, NCH, N), lambda b, g: (b, g, 0, 0)),
        ],
        out_specs=pl.BlockSpec((1, 1, NCH, N), lambda b, g: (b, g, 0, 0)),
        out_shape=jax.ShapeDtypeStruct((B, G, NCH, N), jnp.float32),
    )(score, qn, kn, vn)


def kernel(x, abs_x, idx, Wq, Wk, Wv, Wq_nl, Wk_nl, Wv_nl):
    xr = x.reshape(B, C, NK)
    idx_t = jnp.swapaxes(idx.reshape(B, N, K), 1, 2)
    absr = abs_x.reshape(B, C // 2, N)
    sel = jnp.asarray(np.repeat(np.eye(nb, dtype=np.float32), K, axis=0))
    out_l, attn, idx_safe_t, qn, kn, vn = _stage1(
        xr, idx_t, absr, Wq, Wk, Wv, Wq_nl, Wk_nl, Wv_nl, sel)
    idx_flat = jnp.swapaxes(idx_safe_t, 1, 2).reshape(B, 1, NK)
    score = _stage2(attn, idx_flat)
    out_all = _stage3(score.reshape(B, G, 1, N),
                      qn.reshape(B, G, NCH, N),
                      kn.reshape(B, G, NCH, N),
                      vn.reshape(B, G, NCH, N))
    return jnp.concatenate([out_l.reshape(B, LC, N, 1),
                            out_all.reshape(B, NLC, N, 1)], axis=1)


# speedup vs baseline: 3.8326x; 1.6191x over previous
import functools

import numpy as np
import jax
import jax.numpy as jnp
from jax import lax
from jax.experimental import pallas as pl
from jax.experimental.pallas import tpu as pltpu
from jax.experimental.pallas import tpu_sc as plsc

def _sort_network(n=16):
    comps = []
    length = 1
    while length < n:
        for lo in range(0, n, 2 * length):
            step = length
            while step >= 1:
                if step == length:
                    rng = [(lo + i, lo + 2 * length - 1 - i)
                           for i in range(length)]
                else:
                    rng = [(i, i + step)
                           for i in range(lo, lo + 2 * length - step)
                           if (i - lo) % (2 * step) < step]
                comps.extend(rng)
                step //= 2
        length *= 2
    return comps


_COMPS = _sort_network(16)

B, C, N, K = 2, 256, 2048, 16
G = 4
LC, NLC = 192, 64
GC = LC // G
NCH = NLC // G
NK = N * K
NB = 8
nb = N // NB
NKb = nb * K


def _stage1_body(x_ref, idxt_ref, absx_ref, wq_ref, wk_ref, wv_ref,
                 wqn_ref, wkn_ref, wvn_ref, sel_ref,
                 outl_ref, attn_ref, idxs_ref, qn_ref, kn_ref, vn_ref):
    x2 = x_ref[0]
    sel = sel_ref[...]
    q = jnp.dot(wq_ref[...], x2, preferred_element_type=jnp.float32)
    k = jnp.dot(wk_ref[...], x2, preferred_element_type=jnp.float32)
    v = jnp.dot(wv_ref[...], x2, preferred_element_type=jnp.float32)
    prod = q * k
    out = jnp.concatenate(
        [jnp.sum(prod[g * GC:(g + 1) * GC], axis=0, keepdims=True)
         for g in range(G)], axis=0)
    e = jnp.exp(out)
    den = jnp.dot(e, sel, preferred_element_type=jnp.float32, precision=lax.Precision.HIGHEST)
    den_rep = lax.dot_general(den, sel, (((1,), (1,)), ((), ())),
                              preferred_element_type=jnp.float32, precision=lax.Precision.HIGHEST)
    sm = e / den_rep
    attn_ref[0] = sm
    w = (v.reshape(G, GC, NKb) * sm[:, None, :]).reshape(LC, NKb)
    outl_ref[0] = jnp.dot(w, sel, preferred_element_type=jnp.float32, precision=lax.Precision.HIGHEST)
    idxt = idxt_ref[0]
    a = [idxt[i:i + 1] for i in range(K)]
    p = [jnp.zeros_like(a[0]) + i for i in range(K)]
    for i, l in _COMPS:
        swap = a[i] > a[l]
        a[i], a[l] = (jnp.where(swap, a[l], a[i]),
                      jnp.where(swap, a[i], a[l]))
        p[i], p[l] = (jnp.where(swap, p[l], p[i]),
                      jnp.where(swap, p[i], p[l]))
    ones = jnp.ones_like(a[0], dtype=jnp.bool_)
    keep_s = [(a[t + 1] != a[t]) if t < K - 1 else ones for t in range(K)]
    rows = []
    for i in range(K):
        keep = None
        for t in range(K):
            hit = (p[t] == i) & keep_s[t]
            keep = hit if keep is None else (keep | hit)
        rows.append(jnp.where(keep, idxt[i:i + 1], N + i))
    idxs_ref[0] = jnp.concatenate(rows, axis=0)
    a = absx_ref[0]
    qn_ref[0] = jnp.dot(wqn_ref[...], a, preferred_element_type=jnp.float32, precision=lax.Precision.HIGHEST)
    kn_ref[0] = jnp.dot(wkn_ref[...], a, preferred_element_type=jnp.float32, precision=lax.Precision.HIGHEST)
    vn_ref[0] = jnp.dot(wvn_ref[...], a, preferred_element_type=jnp.float32, precision=lax.Precision.HIGHEST)


def _stage1(xr, idx_t, absr, Wq, Wk, Wv, Wq_nl, Wk_nl, Wv_nl, sel):
    f32 = jnp.float32
    return pl.pallas_call(
        _stage1_body,
        grid=(B, NB),
        in_specs=[
            pl.BlockSpec((1, C, NKb), lambda b, n: (b, 0, n)),
            pl.BlockSpec((1, K, nb), lambda b, n: (b, 0, n)),
            pl.BlockSpec((1, C // 2, nb), lambda b, n: (b, 0, n)),
            pl.BlockSpec((LC, C), lambda b, n: (0, 0)),
            pl.BlockSpec((LC, C), lambda b, n: (0, 0)),
            pl.BlockSpec((LC, C), lambda b, n: (0, 0)),
            pl.BlockSpec((NLC, C // 2), lambda b, n: (0, 0)),
            pl.BlockSpec((NLC, C // 2), lambda b, n: (0, 0)),
            pl.BlockSpec((NLC, C // 2), lambda b, n: (0, 0)),
            pl.BlockSpec((NKb, nb), lambda b, n: (0, 0)),
        ],
        out_specs=[
            pl.BlockSpec((1, LC, nb), lambda b, n: (b, 0, n)),
            pl.BlockSpec((1, G, NKb), lambda b, n: (b, 0, n)),
            pl.BlockSpec((1, K, nb), lambda b, n: (b, 0, n)),
            pl.BlockSpec((1, NLC, nb), lambda b, n: (b, 0, n)),
            pl.BlockSpec((1, NLC, nb), lambda b, n: (b, 0, n)),
            pl.BlockSpec((1, NLC, nb), lambda b, n: (b, 0, n)),
        ],
        out_shape=[
            jax.ShapeDtypeStruct((B, LC, N), f32),
            jax.ShapeDtypeStruct((B, G, NK), f32),
            jax.ShapeDtypeStruct((B, K, N), jnp.int32),
            jax.ShapeDtypeStruct((B, NLC, N), f32),
            jax.ShapeDtypeStruct((B, NLC, N), f32),
            jax.ShapeDtypeStruct((B, NLC, N), f32),
        ],
    )(xr, idx_t, absr, Wq, Wk, Wv, Wq_nl, Wk_nl, Wv_nl, sel)


CNK = 512
NCHUNK = NK // CNK


def _stage2_body(attn_ref, idxf_ref, score_ref):
    ch = pl.program_id(1)
    idx_row = idxf_ref[0]
    idx_col = jnp.transpose(idx_row.reshape(1, CNK), (1, 0))
    iota_m = lax.broadcasted_iota(jnp.int32, (CNK, N), 1)
    oh = (iota_m == idx_col).astype(jnp.float32)
    part = jnp.dot(attn_ref[0], oh, preferred_element_type=jnp.float32, precision=lax.Precision.HIGHEST)

    @pl.when(ch == 0)
    def _():
        score_ref[0] = part

    @pl.when(ch != 0)
    def _():
        score_ref[0] = score_ref[0] + part


def _stage2(attn_flat, idx_flat):
    return pl.pallas_call(
        _stage2_body,
        grid=(B, NCHUNK),
        in_specs=[
            pl.BlockSpec((1, G, CNK), lambda b, ch: (b, 0, ch)),
            pl.BlockSpec((1, 1, CNK), lambda b, ch: (b, 0, ch)),
        ],
        out_specs=pl.BlockSpec((1, G, N), lambda b, ch: (b, 0, 0)),
        out_shape=jax.ShapeDtypeStruct((B, G, N), jnp.float32),
    )(attn_flat, idx_flat)


SP = 4096
NPW = N // 16
SLC = SP // 16
NZ = 2064 // 16


def _stage2_sc_body(attn_hbm, idx_hbm, out_hbm,
                    idx_v, attn_v, score1d, shared, tmp, accv):
    c = lax.axis_index("c")
    s = lax.axis_index("s")
    base = s * NPW
    zero16 = jnp.zeros((16,), jnp.float32)

    def _zero(i, _):
        for g in range(G):
            score1d[pl.ds(g * SP + i * 16, 16)] = zero16
        return 0

    lax.fori_loop(0, NZ, _zero, 0)
    pltpu.sync_copy(idx_hbm.at[c, pl.ds(base, NPW)], idx_v)
    for g in range(G):
        pltpu.sync_copy(attn_hbm.at[c, g, pl.ds(base, NPW)], attn_v)
        gofs = jnp.zeros((16,), jnp.int32) + g * SP

        def _scat(i, _):
            plsc.addupdate_scatter(score1d, [gofs + idx_v[i]], attn_v[i])
            return 0

        lax.fori_loop(0, NPW, _scat, 0)
    pltpu.sync_copy(score1d, shared.at[s, 0])
    plsc.subcore_barrier()
    col = s * SLC
    accs = [[jnp.zeros((16,), jnp.float32) for _ in range(SLC // 16)]
            for _ in range(G)]
    for w2 in range(16):
        for g in range(G):
            pltpu.sync_copy(shared.at[w2, 0, pl.ds(g * SP + col, SLC)], tmp)
            for j in range(SLC // 16):
                accs[g][j] = accs[g][j] + tmp[pl.ds(j * 16, 16)]
    for g in range(G):
        for j in range(SLC // 16):
            accv[pl.ds(j * 16, 16)] = accs[g][j]
        pltpu.sync_copy(accv, out_hbm.at[c, g, pl.ds(col, SLC)])


def _stage2_sc(attn4, idx_safe):
    f32 = jnp.float32
    call = pl.kernel(
        _stage2_sc_body,
        out_type=jax.ShapeDtypeStruct((B, G, SP), f32),
        mesh=plsc.VectorSubcoreMesh(core_axis_name="c", subcore_axis_name="s"),
        compiler_params=pltpu.CompilerParams(needs_layout_passes=False),
        scratch_types=[
            pltpu.VMEM((NPW, K), jnp.int32),
            pltpu.VMEM((NPW, K), f32),
            pltpu.VMEM((G * SP,), f32),
            pltpu.VMEM_SHARED((16, 1, G * SP), f32),
            pltpu.VMEM((SLC,), f32),
            pltpu.VMEM((SLC,), f32),
        ],
    )
    return call(attn4, idx_safe)


def _stage3_body(score_ref, qn_ref, kn_ref, vn_ref, out_ref):
    s = score_ref[0, 0]
    iota = lax.broadcasted_iota(jnp.int32, (1, N), 1)
    oh_rows, vals = [], []
    for _ in range(K):
        m = jnp.max(s)
        j = jnp.min(jnp.where(s == m, iota, N))
        hit = iota == j
        oh_rows.append(hit.astype(jnp.float32))
        vals.append(m.reshape(1, 1))
        s = jnp.where(hit, -3e38, s)
    oh = jnp.concatenate(oh_rows, axis=0)
    val = jnp.concatenate(vals, axis=0)
    qn = qn_ref[0, 0]
    kn = kn_ref[0, 0]
    vn = vn_ref[0, 0]
    k_gT = lax.dot_general(oh, kn, (((1,), (1,)), ((), ())),
                           preferred_element_type=jnp.float32, precision=lax.Precision.HIGHEST)
    v_gT = lax.dot_general(oh, vn, (((1,), (1,)), ((), ())),
                           preferred_element_type=jnp.float32, precision=lax.Precision.HIGHEST)
    v_gT = v_gT * jnp.tanh(val)
    at = jnp.dot(k_gT, qn, preferred_element_type=jnp.float32, precision=lax.Precision.HIGHEST)
    at = at - jnp.max(at, axis=0, keepdims=True)
    e = jnp.exp(at)
    at_sm = e / jnp.sum(e, axis=0, keepdims=True)
    out_ref[0, 0] = lax.dot_general(
        v_gT, at_sm, (((0,), (0,)), ((), ())),
        preferred_element_type=jnp.float32, precision=lax.Precision.HIGHEST)


def _stage3(score, qn, kn, vn):
    return pl.pallas_call(
        _stage3_body,
        grid=(B, G),
        in_specs=[
            pl.BlockSpec((1, 1, 1, N), lambda b, g: (b, g, 0, 0)),
            pl.BlockSpec((1, 1, NCH, N), lambda b, g: (b, g, 0, 0)),
            pl.BlockSpec((1, 1, NCH, N), lambda b, g: (b, g, 0, 0)),
            pl.BlockSpec((1, 1, NCH, N), lambda b, g: (b, g, 0, 0)),
        ],
        out_specs=pl.BlockSpec((1, 1, NCH, N), lambda b, g: (b, g, 0, 0)),
        out_shape=jax.ShapeDtypeStruct((B, G, NCH, N), jnp.float32),
    )(score, qn, kn, vn)


def kernel(x, abs_x, idx, Wq, Wk, Wv, Wq_nl, Wk_nl, Wv_nl):
    xr = x.reshape(B, C, NK)
    idx_t = jnp.swapaxes(idx.reshape(B, N, K), 1, 2)
    absr = abs_x.reshape(B, C // 2, N)
    sel = jnp.asarray(np.repeat(np.eye(nb, dtype=np.float32), K, axis=0))
    out_l, attn, idx_safe_t, qn, kn, vn = _stage1(
        xr, idx_t, absr, Wq, Wk, Wv, Wq_nl, Wk_nl, Wv_nl, sel)
    idx_safe = jnp.swapaxes(idx_safe_t, 1, 2)
    score_p = _stage2_sc(attn.reshape(B, G, N, K), idx_safe)
    score = score_p[:, :, :N]
    out_all = _stage3(score.reshape(B, G, 1, N),
                      qn.reshape(B, G, NCH, N),
                      kn.reshape(B, G, NCH, N),
                      vn.reshape(B, G, NCH, N))
    return jnp.concatenate([out_l.reshape(B, LC, N, 1),
                            out_all.reshape(B, NLC, N, 1)], axis=1)


# speedup vs baseline: 5.0731x; 1.3237x over previous
import functools

import numpy as np
import jax
import jax.numpy as jnp
from jax import lax
from jax.experimental import pallas as pl
from jax.experimental.pallas import tpu as pltpu
from jax.experimental.pallas import tpu_sc as plsc

def _sort_network(n=16):
    comps = []
    length = 1
    while length < n:
        for lo in range(0, n, 2 * length):
            step = length
            while step >= 1:
                if step == length:
                    rng = [(lo + i, lo + 2 * length - 1 - i)
                           for i in range(length)]
                else:
                    rng = [(i, i + step)
                           for i in range(lo, lo + 2 * length - step)
                           if (i - lo) % (2 * step) < step]
                comps.extend(rng)
                step //= 2
        length *= 2
    return comps


_COMPS = _sort_network(16)

B, C, N, K = 2, 256, 2048, 16
G = 4
LC, NLC = 192, 64
GC = LC // G
NCH = NLC // G
NK = N * K
NB = 8
nb = N // NB
NKb = nb * K


def _stage1_body(x_ref, idxt_ref, absx_ref, wq_ref, wk_ref, wv_ref,
                 wqn_ref, wkn_ref, wvn_ref, sel_ref,
                 outl_ref, attn_ref, idxs_ref, qn_ref, kn_ref, vn_ref):
    x2 = x_ref[0]
    sel = sel_ref[...]
    q = jnp.dot(wq_ref[...], x2, preferred_element_type=jnp.float32)
    k = jnp.dot(wk_ref[...], x2, preferred_element_type=jnp.float32)
    v = jnp.dot(wv_ref[...], x2, preferred_element_type=jnp.float32)
    prod = q * k
    out = jnp.concatenate(
        [jnp.sum(prod[g * GC:(g + 1) * GC], axis=0, keepdims=True)
         for g in range(G)], axis=0)
    e = jnp.exp(out)
    den = jnp.dot(e, sel, preferred_element_type=jnp.float32, precision=lax.Precision.HIGHEST)
    den_rep = jnp.broadcast_to(den[:, :, None], (G, nb, K)).reshape(G, NKb)
    sm = e / den_rep
    attn_ref[0] = sm
    w = (v.reshape(G, GC, NKb) * sm[:, None, :]).reshape(LC, NKb)
    outl_ref[0] = jnp.dot(w, sel, preferred_element_type=jnp.float32)
    idxt = idxt_ref[0]
    a = [idxt[i:i + 1] for i in range(K)]
    p = [jnp.zeros_like(a[0]) + i for i in range(K)]
    for i, l in _COMPS:
        swap = a[i] > a[l]
        a[i], a[l] = (jnp.where(swap, a[l], a[i]),
                      jnp.where(swap, a[i], a[l]))
        p[i], p[l] = (jnp.where(swap, p[l], p[i]),
                      jnp.where(swap, p[i], p[l]))
    ones = jnp.ones_like(a[0], dtype=jnp.bool_)
    keep_s = [(a[t + 1] != a[t]) if t < K - 1 else ones for t in range(K)]
    rows = []
    for i in range(K):
        keep = None
        for t in range(K):
            hit = (p[t] == i) & keep_s[t]
            keep = hit if keep is None else (keep | hit)
        rows.append(jnp.where(keep, idxt[i:i + 1], N + i))
    idxs_ref[0] = jnp.concatenate(rows, axis=0)
    a = absx_ref[0]
    qn_ref[0] = jnp.dot(wqn_ref[...], a, preferred_element_type=jnp.float32, precision=lax.Precision.HIGHEST)
    kn_ref[0] = jnp.dot(wkn_ref[...], a, preferred_element_type=jnp.float32, precision=lax.Precision.HIGHEST)
    vn_ref[0] = jnp.dot(wvn_ref[...], a, preferred_element_type=jnp.float32, precision=lax.Precision.HIGHEST)


def _stage1(xr, idx_t, absr, Wq, Wk, Wv, Wq_nl, Wk_nl, Wv_nl, sel):
    f32 = jnp.float32
    return pl.pallas_call(
        _stage1_body,
        grid=(B, NB),
        in_specs=[
            pl.BlockSpec((1, C, NKb), lambda b, n: (b, 0, n)),
            pl.BlockSpec((1, K, nb), lambda b, n: (b, 0, n)),
            pl.BlockSpec((1, C // 2, nb), lambda b, n: (b, 0, n)),
            pl.BlockSpec((LC, C), lambda b, n: (0, 0)),
            pl.BlockSpec((LC, C), lambda b, n: (0, 0)),
            pl.BlockSpec((LC, C), lambda b, n: (0, 0)),
            pl.BlockSpec((NLC, C // 2), lambda b, n: (0, 0)),
            pl.BlockSpec((NLC, C // 2), lambda b, n: (0, 0)),
            pl.BlockSpec((NLC, C // 2), lambda b, n: (0, 0)),
            pl.BlockSpec((NKb, nb), lambda b, n: (0, 0)),
        ],
        out_specs=[
            pl.BlockSpec((1, LC, nb), lambda b, n: (b, 0, n)),
            pl.BlockSpec((1, G, NKb), lambda b, n: (b, 0, n)),
            pl.BlockSpec((1, K, nb), lambda b, n: (b, 0, n)),
            pl.BlockSpec((1, NLC, nb), lambda b, n: (b, 0, n)),
            pl.BlockSpec((1, NLC, nb), lambda b, n: (b, 0, n)),
            pl.BlockSpec((1, NLC, nb), lambda b, n: (b, 0, n)),
        ],
        out_shape=[
            jax.ShapeDtypeStruct((B, LC, N), f32),
            jax.ShapeDtypeStruct((B, G, NK), f32),
            jax.ShapeDtypeStruct((B, K, N), jnp.int32),
            jax.ShapeDtypeStruct((B, NLC, N), f32),
            jax.ShapeDtypeStruct((B, NLC, N), f32),
            jax.ShapeDtypeStruct((B, NLC, N), f32),
        ],
    )(xr, idx_t, absr, Wq, Wk, Wv, Wq_nl, Wk_nl, Wv_nl, sel)


CNK = 512
NCHUNK = NK // CNK


def _stage2_body(attn_ref, idxf_ref, score_ref):
    ch = pl.program_id(1)
    idx_row = idxf_ref[0]
    idx_col = jnp.transpose(idx_row.reshape(1, CNK), (1, 0))
    iota_m = lax.broadcasted_iota(jnp.int32, (CNK, N), 1)
    oh = (iota_m == idx_col).astype(jnp.float32)
    part = jnp.dot(attn_ref[0], oh, preferred_element_type=jnp.float32, precision=lax.Precision.HIGHEST)

    @pl.when(ch == 0)
    def _():
        score_ref[0] = part

    @pl.when(ch != 0)
    def _():
        score_ref[0] = score_ref[0] + part


def _stage2(attn_flat, idx_flat):
    return pl.pallas_call(
        _stage2_body,
        grid=(B, NCHUNK),
        in_specs=[
            pl.BlockSpec((1, G, CNK), lambda b, ch: (b, 0, ch)),
            pl.BlockSpec((1, 1, CNK), lambda b, ch: (b, 0, ch)),
        ],
        out_specs=pl.BlockSpec((1, G, N), lambda b, ch: (b, 0, 0)),
        out_shape=jax.ShapeDtypeStruct((B, G, N), jnp.float32),
    )(attn_flat, idx_flat)


SP = 4096
NPW = N // 16
SLC = SP // 16
NZ = 2064 // 16


def _stage2_sc_body(attn_hbm, idx_hbm, out_hbm,
                    idx_v, attn_v, score1d, shared, tmp, accv):
    c = lax.axis_index("c")
    s = lax.axis_index("s")
    base = s * NPW
    zero16 = jnp.zeros((16,), jnp.float32)

    def _zero(i, _):
        for g in range(G):
            score1d[pl.ds(g * SP + i * 16, 16)] = zero16
        return 0

    lax.fori_loop(0, NZ, _zero, 0)
    pltpu.sync_copy(idx_hbm.at[c, pl.ds(base, NPW)], idx_v)
    for g in range(G):
        pltpu.sync_copy(attn_hbm.at[c, g, pl.ds(base, NPW)], attn_v)
        gofs = jnp.zeros((16,), jnp.int32) + g * SP

        def _scat(i, _):
            plsc.addupdate_scatter(score1d, [gofs + idx_v[i]], attn_v[i])
            return 0

        lax.fori_loop(0, NPW, _scat, 0)
    pltpu.sync_copy(score1d, shared.at[s, 0])
    plsc.subcore_barrier()
    col = s * SLC
    accs = [[jnp.zeros((16,), jnp.float32) for _ in range(SLC // 16)]
            for _ in range(G)]
    for w2 in range(16):
        for g in range(G):
            pltpu.sync_copy(shared.at[w2, 0, pl.ds(g * SP + col, SLC)], tmp)
            for j in range(SLC // 16):
                accs[g][j] = accs[g][j] + tmp[pl.ds(j * 16, 16)]
    for g in range(G):
        for j in range(SLC // 16):
            accv[pl.ds(j * 16, 16)] = accs[g][j]
        pltpu.sync_copy(accv, out_hbm.at[c, g, pl.ds(col, SLC)])


def _stage2_sc(attn4, idx_safe):
    f32 = jnp.float32
    call = pl.kernel(
        _stage2_sc_body,
        out_type=jax.ShapeDtypeStruct((B, G, SP), f32),
        mesh=plsc.VectorSubcoreMesh(core_axis_name="c", subcore_axis_name="s"),
        compiler_params=pltpu.CompilerParams(needs_layout_passes=False),
        scratch_types=[
            pltpu.VMEM((NPW, K), jnp.int32),
            pltpu.VMEM((NPW, K), f32),
            pltpu.VMEM((G * SP,), f32),
            pltpu.VMEM_SHARED((16, 1, G * SP), f32),
            pltpu.VMEM((SLC,), f32),
            pltpu.VMEM((SLC,), f32),
        ],
    )
    return call(attn4, idx_safe)


def _stage3_body(score_ref, qn_ref, kn_ref, vn_ref, out_ref):
    s = score_ref[0, 0]
    iota = lax.broadcasted_iota(jnp.int32, (1, N), 1)
    oh_rows, vals = [], []
    for _ in range(K):
        m = jnp.max(s)
        j = jnp.min(jnp.where(s == m, iota, N))
        hit = iota == j
        oh_rows.append(hit.astype(jnp.float32))
        vals.append(m.reshape(1, 1))
        s = jnp.where(hit, -3e38, s)
    oh = jnp.concatenate(oh_rows, axis=0)
    val = jnp.concatenate(vals, axis=0)
    qn = qn_ref[0, 0]
    kn = kn_ref[0, 0]
    vn = vn_ref[0, 0]
    k_gT = lax.dot_general(oh, kn, (((1,), (1,)), ((), ())),
                           preferred_element_type=jnp.float32, precision=lax.Precision.HIGHEST)
    v_gT = lax.dot_general(oh, vn, (((1,), (1,)), ((), ())),
                           preferred_element_type=jnp.float32, precision=lax.Precision.HIGHEST)
    v_gT = v_gT * jnp.tanh(val)
    at = jnp.dot(k_gT, qn, preferred_element_type=jnp.float32, precision=lax.Precision.HIGHEST)
    at = at - jnp.max(at, axis=0, keepdims=True)
    e = jnp.exp(at)
    at_sm = e / jnp.sum(e, axis=0, keepdims=True)
    out_ref[0, 0] = lax.dot_general(
        v_gT, at_sm, (((0,), (0,)), ((), ())),
        preferred_element_type=jnp.float32, precision=lax.Precision.HIGHEST)


def _stage3(score, qn, kn, vn):
    return pl.pallas_call(
        _stage3_body,
        grid=(B, G),
        in_specs=[
            pl.BlockSpec((1, 1, 1, N), lambda b, g: (b, g, 0, 0)),
            pl.BlockSpec((1, 1, NCH, N), lambda b, g: (b, g, 0, 0)),
            pl.BlockSpec((1, 1, NCH, N), lambda b, g: (b, g, 0, 0)),
            pl.BlockSpec((1, 1, NCH, N), lambda b, g: (b, g, 0, 0)),
        ],
        out_specs=pl.BlockSpec((1, 1, NCH, N), lambda b, g: (b, g, 0, 0)),
        out_shape=jax.ShapeDtypeStruct((B, G, NCH, N), jnp.float32),
    )(score, qn, kn, vn)


def kernel(x, abs_x, idx, Wq, Wk, Wv, Wq_nl, Wk_nl, Wv_nl):
    xr = x.reshape(B, C, NK)
    idx_t = jnp.swapaxes(idx.reshape(B, N, K), 1, 2)
    absr = abs_x.reshape(B, C // 2, N)
    sel = jnp.asarray(np.repeat(np.eye(nb, dtype=np.float32), K, axis=0))
    out_l, attn, idx_safe_t, qn, kn, vn = _stage1(
        xr, idx_t, absr, Wq, Wk, Wv, Wq_nl, Wk_nl, Wv_nl, sel)
    idx_safe = jnp.swapaxes(idx_safe_t, 1, 2)
    score_p = _stage2_sc(attn.reshape(B, G, N, K), idx_safe)
    score = score_p[:, :, :N]
    out_all = _stage3(score.reshape(B, G, 1, N),
                      qn.reshape(B, G, NCH, N),
                      kn.reshape(B, G, NCH, N),
                      vn.reshape(B, G, NCH, N))
    return jnp.concatenate([out_l.reshape(B, LC, N, 1),
                            out_all.reshape(B, NLC, N, 1)], axis=1)


# speedup vs baseline: 5.1160x; 1.0085x over previous
import functools

import numpy as np
import jax
import jax.numpy as jnp
from jax import lax
from jax.experimental import pallas as pl
from jax.experimental.pallas import tpu as pltpu
from jax.experimental.pallas import tpu_sc as plsc

def _sort_network(n=16):
    comps = []
    length = 1
    while length < n:
        for lo in range(0, n, 2 * length):
            step = length
            while step >= 1:
                if step == length:
                    rng = [(lo + i, lo + 2 * length - 1 - i)
                           for i in range(length)]
                else:
                    rng = [(i, i + step)
                           for i in range(lo, lo + 2 * length - step)
                           if (i - lo) % (2 * step) < step]
                comps.extend(rng)
                step //= 2
        length *= 2
    return comps


_COMPS = _sort_network(16)

B, C, N, K = 2, 256, 2048, 16
G = 4
LC, NLC = 192, 64
GC = LC // G
NCH = NLC // G
NK = N * K
NB = 8
nb = N // NB
NKb = nb * K


def _stage1_body(x_ref, idxt_ref, absx_ref, wq_ref, wk_ref, wv_ref,
                 wqn_ref, wkn_ref, wvn_ref, sel_ref,
                 outl_ref, attn_ref, idxs_ref, qn_ref, kn_ref, vn_ref):
    x2 = x_ref[0]
    sel = sel_ref[...]
    q = jnp.dot(wq_ref[...], x2, preferred_element_type=jnp.float32)
    k = jnp.dot(wk_ref[...], x2, preferred_element_type=jnp.float32)
    v = jnp.dot(wv_ref[...], x2, preferred_element_type=jnp.float32)
    prod = q * k
    out = jnp.concatenate(
        [jnp.sum(prod[g * GC:(g + 1) * GC], axis=0, keepdims=True)
         for g in range(G)], axis=0)
    e = jnp.exp(out)
    den = jnp.dot(e, sel, preferred_element_type=jnp.float32, precision=lax.Precision.HIGHEST)
    den_rep = jnp.broadcast_to(den[:, :, None], (G, nb, K)).reshape(G, NKb)
    sm = e / den_rep
    attn_ref[0] = sm
    w = (v.reshape(G, GC, NKb) * sm[:, None, :]).reshape(LC, NKb)
    outl_ref[0] = jnp.dot(w, sel, preferred_element_type=jnp.float32)
    idxt = idxt_ref[0]
    a = [idxt[i:i + 1] for i in range(K)]
    p = [jnp.zeros_like(a[0]) + i for i in range(K)]
    for i, l in _COMPS:
        swap = a[i] > a[l]
        a[i], a[l] = (jnp.where(swap, a[l], a[i]),
                      jnp.where(swap, a[i], a[l]))
        p[i], p[l] = (jnp.where(swap, p[l], p[i]),
                      jnp.where(swap, p[i], p[l]))
    ones = jnp.ones_like(a[0], dtype=jnp.bool_)
    keep_s = [(a[t + 1] != a[t]) if t < K - 1 else ones for t in range(K)]
    rows = []
    for i in range(K):
        keep = None
        for t in range(K):
            hit = (p[t] == i) & keep_s[t]
            keep = hit if keep is None else (keep | hit)
        rows.append(jnp.where(keep, idxt[i:i + 1], N + i))
    idxs_ref[0] = jnp.concatenate(rows, axis=0)
    a = absx_ref[0]
    qn_ref[0] = jnp.dot(wqn_ref[...], a, preferred_element_type=jnp.float32, precision=lax.Precision.HIGHEST)
    kn_ref[0] = jnp.dot(wkn_ref[...], a, preferred_element_type=jnp.float32, precision=lax.Precision.HIGHEST)
    vn_ref[0] = jnp.dot(wvn_ref[...], a, preferred_element_type=jnp.float32, precision=lax.Precision.HIGHEST)


def _stage1(xr, idx_t, absr, Wq, Wk, Wv, Wq_nl, Wk_nl, Wv_nl, sel):
    f32 = jnp.float32
    return pl.pallas_call(
        _stage1_body,
        grid=(B, NB),
        in_specs=[
            pl.BlockSpec((1, C, NKb), lambda b, n: (b, 0, n)),
            pl.BlockSpec((1, K, nb), lambda b, n: (b, 0, n)),
            pl.BlockSpec((1, C // 2, nb), lambda b, n: (b, 0, n)),
            pl.BlockSpec((LC, C), lambda b, n: (0, 0)),
            pl.BlockSpec((LC, C), lambda b, n: (0, 0)),
            pl.BlockSpec((LC, C), lambda b, n: (0, 0)),
            pl.BlockSpec((NLC, C // 2), lambda b, n: (0, 0)),
            pl.BlockSpec((NLC, C // 2), lambda b, n: (0, 0)),
            pl.BlockSpec((NLC, C // 2), lambda b, n: (0, 0)),
            pl.BlockSpec((NKb, nb), lambda b, n: (0, 0)),
        ],
        out_specs=[
            pl.BlockSpec((1, LC, nb), lambda b, n: (b, 0, n)),
            pl.BlockSpec((1, G, NKb), lambda b, n: (b, 0, n)),
            pl.BlockSpec((1, K, nb), lambda b, n: (b, 0, n)),
            pl.BlockSpec((1, NLC, nb), lambda b, n: (b, 0, n)),
            pl.BlockSpec((1, NLC, nb), lambda b, n: (b, 0, n)),
            pl.BlockSpec((1, NLC, nb), lambda b, n: (b, 0, n)),
        ],
        out_shape=[
            jax.ShapeDtypeStruct((B, LC, N), f32),
            jax.ShapeDtypeStruct((B, G, NK), f32),
            jax.ShapeDtypeStruct((B, K, N), jnp.int32),
            jax.ShapeDtypeStruct((B, NLC, N), f32),
            jax.ShapeDtypeStruct((B, NLC, N), f32),
            jax.ShapeDtypeStruct((B, NLC, N), f32),
        ],
    )(xr, idx_t, absr, Wq, Wk, Wv, Wq_nl, Wk_nl, Wv_nl, sel)


SP = 4096
NPW = N // 16
SLC = SP // 16
NZ = 2064 // 16


def _stage2_sc_body(attn_hbm, idx_hbm, out_hbm,
                    idx_v, attn_v, score1d, shared, tmp4, accv):
    c = lax.axis_index("c")
    s = lax.axis_index("s")
    base = s * NPW
    zero16 = jnp.zeros((16,), jnp.float32)

    def _zero(i, _):
        for g in range(G):
            score1d[pl.ds(g * SP + i * 16, 16)] = zero16
        return 0

    lax.fori_loop(0, NZ, _zero, 0)
    pltpu.sync_copy(idx_hbm.at[c, pl.ds(base, NPW)], idx_v)
    for g in range(G):
        pltpu.sync_copy(attn_hbm.at[c, g, pl.ds(base, NPW)], attn_v)
        gofs = jnp.zeros((16,), jnp.int32) + g * SP

        def _scat(i, _):
            plsc.addupdate_scatter(score1d, [gofs + idx_v[i]], attn_v[i])
            return 0

        lax.fori_loop(0, NPW, _scat, 0)
    for g in range(G):
        pltpu.sync_copy(score1d.at[pl.ds(g * SP, SP)], shared.at[s, g, 0])
    plsc.subcore_barrier()
    col = s * SLC
    accs = [[jnp.zeros((16,), jnp.float32) for _ in range(SLC // 16)]
            for _ in range(G)]
    for w2 in range(16):
        pltpu.sync_copy(shared.at[w2, :, 0, pl.ds(col, SLC)], tmp4)
        for g in range(G):
            for j in range(SLC // 16):
                accs[g][j] = accs[g][j] + tmp4[g, pl.ds(j * 16, 16)]
    for g in range(G):
        for j in range(SLC // 16):
            accv[pl.ds(j * 16, 16)] = accs[g][j]
        pltpu.sync_copy(accv, out_hbm.at[c, g, pl.ds(col, SLC)])


def _stage2_sc(attn4, idx_safe):
    f32 = jnp.float32
    call = pl.kernel(
        _stage2_sc_body,
        out_type=jax.ShapeDtypeStruct((B, G, SP), f32),
        mesh=plsc.VectorSubcoreMesh(core_axis_name="c", subcore_axis_name="s"),
        compiler_params=pltpu.CompilerParams(needs_layout_passes=False),
        scratch_types=[
            pltpu.VMEM((NPW, K), jnp.int32),
            pltpu.VMEM((NPW, K), f32),
            pltpu.VMEM((G * SP,), f32),
            pltpu.VMEM_SHARED((16, G, 1, SP), f32),
            pltpu.VMEM((G, SLC), f32),
            pltpu.VMEM((SLC,), f32),
        ],
    )
    return call(attn4, idx_safe)


def _stage3_body(score_ref, qn_ref, kn_ref, vn_ref, out_ref):
    s = score_ref[0, 0]
    iota = lax.broadcasted_iota(jnp.int32, (1, N), 1)
    oh_rows, vals = [], []
    for _ in range(K):
        m = jnp.max(s)
        j = jnp.min(jnp.where(s == m, iota, N))
        hit = iota == j
        oh_rows.append(hit.astype(jnp.float32))
        vals.append(m.reshape(1, 1))
        s = jnp.where(hit, -3e38, s)
    oh = jnp.concatenate(oh_rows, axis=0)
    val = jnp.concatenate(vals, axis=0)
    qn = qn_ref[0, 0]
    kn = kn_ref[0, 0]
    vn = vn_ref[0, 0]
    k_gT = lax.dot_general(oh, kn, (((1,), (1,)), ((), ())),
                           preferred_element_type=jnp.float32, precision=lax.Precision.HIGHEST)
    v_gT = lax.dot_general(oh, vn, (((1,), (1,)), ((), ())),
                           preferred_element_type=jnp.float32, precision=lax.Precision.HIGHEST)
    v_gT = v_gT * jnp.tanh(val)
    at = jnp.dot(k_gT, qn, preferred_element_type=jnp.float32, precision=lax.Precision.HIGHEST)
    at = at - jnp.max(at, axis=0, keepdims=True)
    e = jnp.exp(at)
    at_sm = e / jnp.sum(e, axis=0, keepdims=True)
    out_ref[0, 0] = lax.dot_general(
        v_gT, at_sm, (((0,), (0,)), ((), ())),
        preferred_element_type=jnp.float32, precision=lax.Precision.HIGHEST)


def _stage3(score, qn, kn, vn):
    return pl.pallas_call(
        _stage3_body,
        grid=(B, G),
        in_specs=[
            pl.BlockSpec((1, 1, 1, N), lambda b, g: (b, g, 0, 0)),
            pl.BlockSpec((1, 1, NCH, N), lambda b, g: (b, g, 0, 0)),
            pl.BlockSpec((1, 1, NCH, N), lambda b, g: (b, g, 0, 0)),
            pl.BlockSpec((1, 1, NCH, N), lambda b, g: (b, g, 0, 0)),
        ],
        out_specs=pl.BlockSpec((1, 1, NCH, N), lambda b, g: (b, g, 0, 0)),
        out_shape=jax.ShapeDtypeStruct((B, G, NCH, N), jnp.float32),
    )(score, qn, kn, vn)


def kernel(x, abs_x, idx, Wq, Wk, Wv, Wq_nl, Wk_nl, Wv_nl):
    xr = x.reshape(B, C, NK)
    idx_t = jnp.swapaxes(idx.reshape(B, N, K), 1, 2)
    absr = abs_x.reshape(B, C // 2, N)
    sel = jnp.asarray(np.repeat(np.eye(nb, dtype=np.float32), K, axis=0))
    out_l, attn, idx_safe_t, qn, kn, vn = _stage1(
        xr, idx_t, absr, Wq, Wk, Wv, Wq_nl, Wk_nl, Wv_nl, sel)
    idx_safe = jnp.swapaxes(idx_safe_t, 1, 2)
    score_p = _stage2_sc(attn.reshape(B, G, N, K), idx_safe)
    score = score_p[:, :, :N]
    out_all = _stage3(score.reshape(B, G, 1, N),
                      qn.reshape(B, G, NCH, N),
                      kn.reshape(B, G, NCH, N),
                      vn.reshape(B, G, NCH, N))
    return jnp.concatenate([out_l.reshape(B, LC, N, 1),
                            out_all.reshape(B, NLC, N, 1)], axis=1)
